# Initial kernel scaffold; baseline (speedup 1.0000x reference)
import jax
import jax.numpy as jnp
from jax.experimental import pallas as pl


def kernel(x, edge_index, edge_attr, Wf1, bf1, Ws1, bs1, g1, be1, Wf2, bf2, Ws2, bs2, g2, be2):
    raise NotImplementedError("write your pallas kernel here")


# speedup vs baseline: 1.1528x; 1.1528x over previous
import functools

import jax
import jax.numpy as jnp
from jax import lax
from jax.experimental import pallas as pl
from jax.experimental.pallas import tpu as pltpu
from jax.experimental.pallas import tpu_sc as plsc

N = 10000
E = 320000
D = 128
DE = 16
Z2 = 2 * D

_NC = 2
_NS = 16
_NW = _NC * _NS
_EPW = E // _NW
_B = 80
_GBLK = _EPW // _B
_EPT = E // _NS
_SBLK = _EPT // _B
_QN = N // 4
_QR = _QN + 12


def _sc_gather(tu, tv, src, dst):
    mesh = plsc.VectorSubcoreMesh(core_axis_name="c", subcore_axis_name="s")

    @functools.partial(
        pl.kernel,
        out_type=(
            jax.ShapeDtypeStruct((E, Z2), jnp.float32),
            jax.ShapeDtypeStruct((E, Z2), jnp.float32),
        ),
        mesh=mesh,
        scratch_types=[
            pltpu.VMEM((_B,), jnp.int32),
            pltpu.VMEM((_B,), jnp.int32),
            pltpu.VMEM((_B, Z2), jnp.float32),
            pltpu.VMEM((_B, Z2), jnp.float32),
        ],
    )
    def k(tu_h, tv_h, src_h, dst_h, gu_out, gv_out, idxd, idxs, bu, bv):
        cid = lax.axis_index("c")
        sid = lax.axis_index("s")
        wid = sid * _NC + cid
        ebase = wid * _EPW

        @pl.loop(0, _GBLK)
        def blk(b):
            off = ebase + b * _B
            pltpu.sync_copy(dst_h.at[pl.ds(off, _B)], idxd)
            pltpu.sync_copy(src_h.at[pl.ds(off, _B)], idxs)
            pltpu.sync_copy(tu_h.at[idxd], bu)
            pltpu.sync_copy(bu, gu_out.at[pl.ds(off, _B)])
            pltpu.sync_copy(tv_h.at[idxs], bv)
            pltpu.sync_copy(bv, gv_out.at[pl.ds(off, _B)])

    return k(tu, tv, src, dst)


def _sc_scatter(msg, ix):
    mesh = plsc.VectorSubcoreMesh(core_axis_name="c", subcore_axis_name="s")

    @functools.partial(
        pl.kernel,
        out_type=jax.ShapeDtypeStruct((_NC * _QR, D), jnp.float32),
        mesh=mesh,
        scratch_types=[
            pltpu.VMEM((_B,), jnp.int32),
            pltpu.VMEM((_B, D), jnp.float32),
            pltpu.VMEM((400, D), jnp.float32),
            pltpu.VMEM_SHARED((_QR, D), jnp.float32),
        ],
    )
    def k(msg_h, ix_h, agg_out, idxb, mb, big, agg_sh):
        cid = lax.axis_index("c")
        sid = lax.axis_index("s")

        zero16 = jnp.zeros((16,), jnp.float32)

        def zrow(r, carry):
            for j in range(D // 16):
                big[r, pl.ds(j * 16, 16)] = zero16
            return carry

        lax.fori_loop(0, 400, zrow, 0)

        for r in range(0, _QR, 400):
            sz = min(400, _QR - r)
            pltpu.sync_copy(big.at[pl.ds(0, sz)], agg_sh.at[pl.ds(r, sz)])

        plsc.subcore_barrier()

        ebase = sid * _EPT

        @pl.loop(0, _SBLK)
        def blk(b):
            off = ebase + b * _B
            pltpu.sync_copy(ix_h.at[pl.ds(cid * E + off, _B)], idxb)
            pltpu.sync_copy(msg_h.at[pl.ds(off, _B)], mb)
            pltpu.sync_copy(mb, agg_sh.at[idxb], add=True)

        plsc.subcore_barrier()

        obase = cid * _QR

        for r in range(0, _QR, 400):
            sz = min(400, _QR - r)
            pltpu.sync_copy(agg_sh.at[pl.ds(r, sz)], big.at[pl.ds(0, sz)])
            pltpu.sync_copy(big.at[pl.ds(0, sz)],
                            agg_out.at[pl.ds(obase + r, sz)])

    return k(msg, ix)


def _proj_tc(h, wu, wv):
    def body(h_ref, wu_ref, wv_ref, tu_ref, tv_ref):
        hh = h_ref[...]
        tu_ref[...] = jnp.dot(hh, wu_ref[...], preferred_element_type=jnp.float32)
        tv_ref[...] = jnp.dot(hh, wv_ref[...], preferred_element_type=jnp.float32)

    sds = jax.ShapeDtypeStruct((N, Z2), jnp.float32)
    return pl.pallas_call(body, out_shape=(sds, sds))(h, wu, wv)


def _eproj_tc(ea, we, be):
    blk = 10000

    def body(ea_ref, we_ref, be_ref, out_ref):
        out_ref[...] = (
            jnp.dot(ea_ref[...], we_ref[...], preferred_element_type=jnp.float32)
            + be_ref[...]
        )

    return pl.pallas_call(
        body,
        grid=(E // blk,),
        in_specs=[
            pl.BlockSpec((blk, DE), lambda i: (i, 0)),
            pl.BlockSpec((DE, Z2), lambda i: (0, 0)),
            pl.BlockSpec((1, Z2), lambda i: (0, 0)),
        ],
        out_specs=pl.BlockSpec((blk, Z2), lambda i: (i, 0)),
        out_shape=jax.ShapeDtypeStruct((E, Z2), jnp.float32),
    )(ea, we, be)


def _msg_tc(gu, gv, et):
    blk = 8000

    def body(gu_ref, gv_ref, et_ref, out_ref):
        s = gu_ref[...] + gv_ref[...] + et_ref[...]
        gate = jax.nn.sigmoid(s[:, :D])
        core = jax.nn.softplus(s[:, D:])
        out_ref[...] = gate * core

    return pl.pallas_call(
        body,
        grid=(E // blk,),
        in_specs=[
            pl.BlockSpec((blk, Z2), lambda i: (i, 0)),
            pl.BlockSpec((blk, Z2), lambda i: (i, 0)),
            pl.BlockSpec((blk, Z2), lambda i: (i, 0)),
        ],
        out_specs=pl.BlockSpec((blk, D), lambda i: (i, 0)),
        out_shape=jax.ShapeDtypeStruct((E, D), jnp.float32),
    )(gu, gv, et)


def _idx_tc(dst2d):
    def body(d_ref, o01_ref, o23_ref):
        d = d_ref[...]

        def loc(lo):
            ok = (d >= lo) & (d < lo + _QN)
            return jnp.where(ok, d - lo, _QN)

        o01_ref[...] = jnp.concatenate([loc(0), loc(_QN)], axis=0)
        o23_ref[...] = jnp.concatenate([loc(2 * _QN), loc(3 * _QN)], axis=0)

    sds = jax.ShapeDtypeStruct((2 * E // 128, 128), jnp.int32)
    return pl.pallas_call(body, out_shape=(sds, sds))(dst2d)


def _bn_tc(s0, s1, c0, c1, h, gamma, beta):
    def body(s0_ref, s1_ref, c0_ref, c1_ref, h_ref, g_ref, b_ref, o_ref):
        def asm(a_ref):
            return jnp.concatenate(
                [a_ref[0:_QN, :], a_ref[_QR:_QR + _QN, :]], axis=0)

        y = jnp.concatenate([asm(s0_ref), asm(s1_ref)], axis=0)
        cnt = jnp.concatenate(
            [c0_ref[0:_QN, 0:1], c0_ref[_QR:_QR + _QN, 0:1],
             c1_ref[0:_QN, 0:1], c1_ref[_QR:_QR + _QN, 0:1]], axis=0)
        y = y / jnp.maximum(cnt, 1.0)
        mu = jnp.mean(y, axis=0, keepdims=True)
        d = y - mu
        var = jnp.mean(d * d, axis=0, keepdims=True)
        o_ref[...] = d * lax.rsqrt(var + 1e-5) * g_ref[...] + b_ref[...] + h_ref[...]

    return pl.pallas_call(
        body,
        out_shape=jax.ShapeDtypeStruct((N, D), jnp.float32),
    )(s0, s1, c0, c1, h, gamma, beta)


def _prep(Wf, Ws, bf, bs):
    wu = jnp.concatenate([Wf[:, :D].T, Ws[:, :D].T], axis=1)
    wv = jnp.concatenate([Wf[:, D:2 * D].T, Ws[:, D:2 * D].T], axis=1)
    we = jnp.concatenate([Wf[:, 2 * D:].T, Ws[:, 2 * D:].T], axis=1)
    be = jnp.concatenate([bf, bs]).reshape(1, Z2)
    return wu, wv, we, be


def _layer(h, et, src, dst, ix01, ix23, c0, c1, w, gamma, beta):
    wu, wv = w[0], w[1]
    tu, tv = _proj_tc(h, wu, wv)
    gu, gv = _sc_gather(tu, tv, src, dst)
    msg = _msg_tc(gu, gv, et)
    s0 = _sc_scatter(msg, ix01)
    s1 = _sc_scatter(msg, ix23)
    return _bn_tc(s0, s1, c0, c1, h, gamma.reshape(1, D), beta.reshape(1, D))


def kernel(x, edge_index, edge_attr, Wf1, bf1, Ws1, bs1, g1, be1,
           Wf2, bf2, Ws2, bs2, g2, be2):
    src = edge_index[0]
    dst = edge_index[1]
    w1 = _prep(Wf1, Ws1, bf1, bs1)
    w2 = _prep(Wf2, Ws2, bf2, bs2)

    et1 = _eproj_tc(edge_attr, w1[2], w1[3])
    et2 = _eproj_tc(edge_attr, w2[2], w2[3])

    ix01, ix23 = _idx_tc(dst.reshape(E // 128, 128))
    ix01 = ix01.reshape(2 * E)
    ix23 = ix23.reshape(2 * E)

    ones = jnp.ones((E, D), jnp.float32)
    c0 = _sc_scatter(ones, ix01)
    c1 = _sc_scatter(ones, ix23)

    h1 = _layer(x, et1, src, dst, ix01, ix23, c0, c1, w1, g1, be1)
    h2 = _layer(h1, et2, src, dst, ix01, ix23, c0, c1, w2, g2, be2)
    return h2


# speedup vs baseline: 1.4590x; 1.2657x over previous
import functools

import jax
import jax.numpy as jnp
from jax import lax
from jax.experimental import pallas as pl
from jax.experimental.pallas import tpu as pltpu
from jax.experimental.pallas import tpu_sc as plsc

N = 10000
E = 320000
D = 128
DE = 16
Z2 = 2 * D

_NC = 2
_NS = 16
_NW = _NC * _NS
_EPW = E // _NW
_B = 200
_GBLK = _EPW // _B
_EPT = E // _NS
_SBLK = _EPT // _B
_QN = N // 4
_QR = _QN + 12


def _sc_gather(tu, tv, src, dst):
    mesh = plsc.VectorSubcoreMesh(core_axis_name="c", subcore_axis_name="s")

    @functools.partial(
        pl.kernel,
        out_type=(
            jax.ShapeDtypeStruct((E, Z2), jnp.float32),
            jax.ShapeDtypeStruct((E, Z2), jnp.float32),
        ),
        mesh=mesh,
        scratch_types=[
            pltpu.VMEM((_B,), jnp.int32),
            pltpu.VMEM((_B,), jnp.int32),
            pltpu.VMEM((_B, Z2), jnp.float32),
            pltpu.VMEM((_B, Z2), jnp.float32),
        ],
    )
    def k(tu_h, tv_h, src_h, dst_h, gu_out, gv_out, idxd, idxs, bu, bv):
        cid = lax.axis_index("c")
        sid = lax.axis_index("s")
        wid = sid * _NC + cid
        ebase = wid * _EPW

        @pl.loop(0, _GBLK)
        def blk(b):
            off = ebase + b * _B
            pltpu.sync_copy(dst_h.at[pl.ds(off, _B)], idxd)
            pltpu.sync_copy(src_h.at[pl.ds(off, _B)], idxs)
            pltpu.sync_copy(tu_h.at[idxd], bu)
            pltpu.sync_copy(bu, gu_out.at[pl.ds(off, _B)])
            pltpu.sync_copy(tv_h.at[idxs], bv)
            pltpu.sync_copy(bv, gv_out.at[pl.ds(off, _B)])

    return k(tu, tv, src, dst)


def _sc_scatter(msg, ix):
    mesh = plsc.VectorSubcoreMesh(core_axis_name="c", subcore_axis_name="s")

    @functools.partial(
        pl.kernel,
        out_type=jax.ShapeDtypeStruct((_NC * _QR, D), jnp.float32),
        mesh=mesh,
        scratch_types=[
            pltpu.VMEM((_B,), jnp.int32),
            pltpu.VMEM((_B, D), jnp.float32),
            pltpu.VMEM((400, D), jnp.float32),
            pltpu.VMEM_SHARED((_QR, D), jnp.float32),
        ],
    )
    def k(msg_h, ix_h, agg_out, idxb, mb, big, agg_sh):
        cid = lax.axis_index("c")
        sid = lax.axis_index("s")

        zero16 = jnp.zeros((16,), jnp.float32)

        def zrow(r, carry):
            for j in range(D // 16):
                big[r, pl.ds(j * 16, 16)] = zero16
            return carry

        lax.fori_loop(0, 400, zrow, 0)

        for r in range(0, _QR, 400):
            sz = min(400, _QR - r)
            pltpu.sync_copy(big.at[pl.ds(0, sz)], agg_sh.at[pl.ds(r, sz)])

        plsc.subcore_barrier()

        ebase = sid * _EPT

        @pl.loop(0, _SBLK)
        def blk(b):
            off = ebase + b * _B
            pltpu.sync_copy(ix_h.at[pl.ds(cid * E + off, _B)], idxb)
            pltpu.sync_copy(msg_h.at[pl.ds(off, _B)], mb)
            pltpu.sync_copy(mb, agg_sh.at[idxb], add=True)

        plsc.subcore_barrier()

        obase = cid * _QR

        for r in range(0, _QR, 400):
            sz = min(400, _QR - r)
            pltpu.sync_copy(agg_sh.at[pl.ds(r, sz)], big.at[pl.ds(0, sz)])
            pltpu.sync_copy(big.at[pl.ds(0, sz)],
                            agg_out.at[pl.ds(obase + r, sz)])

    return k(msg, ix)


def _proj_tc(h, wu, wv):
    def body(h_ref, wu_ref, wv_ref, tu_ref, tv_ref):
        hh = h_ref[...]
        tu_ref[...] = jnp.dot(hh, wu_ref[...], preferred_element_type=jnp.float32)
        tv_ref[...] = jnp.dot(hh, wv_ref[...], preferred_element_type=jnp.float32)

    sds = jax.ShapeDtypeStruct((N, Z2), jnp.float32)
    return pl.pallas_call(body, out_shape=(sds, sds))(h, wu, wv)


def _eproj_tc(ea, we, be):
    blk = 10000

    def body(ea_ref, we_ref, be_ref, out_ref):
        out_ref[...] = (
            jnp.dot(ea_ref[...], we_ref[...], preferred_element_type=jnp.float32)
            + be_ref[...]
        )

    return pl.pallas_call(
        body,
        grid=(E // blk,),
        in_specs=[
            pl.BlockSpec((blk, DE), lambda i: (i, 0)),
            pl.BlockSpec((DE, Z2), lambda i: (0, 0)),
            pl.BlockSpec((1, Z2), lambda i: (0, 0)),
        ],
        out_specs=pl.BlockSpec((blk, Z2), lambda i: (i, 0)),
        out_shape=jax.ShapeDtypeStruct((E, Z2), jnp.float32),
    )(ea, we, be)


def _msg_tc(gu, gv, et):
    blk = 8000

    def body(gu_ref, gv_ref, et_ref, out_ref):
        s = gu_ref[...] + gv_ref[...] + et_ref[...]
        gate = jax.nn.sigmoid(s[:, :D])
        core = jax.nn.softplus(s[:, D:])
        out_ref[...] = gate * core

    return pl.pallas_call(
        body,
        grid=(E // blk,),
        in_specs=[
            pl.BlockSpec((blk, Z2), lambda i: (i, 0)),
            pl.BlockSpec((blk, Z2), lambda i: (i, 0)),
            pl.BlockSpec((blk, Z2), lambda i: (i, 0)),
        ],
        out_specs=pl.BlockSpec((blk, D), lambda i: (i, 0)),
        out_shape=jax.ShapeDtypeStruct((E, D), jnp.float32),
    )(gu, gv, et)


def _idx_tc(dst2d):
    def body(d_ref, o01_ref, o23_ref):
        d = d_ref[...]

        def loc(lo):
            ok = (d >= lo) & (d < lo + _QN)
            return jnp.where(ok, d - lo, _QN)

        o01_ref[...] = jnp.concatenate([loc(0), loc(_QN)], axis=0)
        o23_ref[...] = jnp.concatenate([loc(2 * _QN), loc(3 * _QN)], axis=0)

    sds = jax.ShapeDtypeStruct((2 * E // 128, 128), jnp.int32)
    return pl.pallas_call(body, out_shape=(sds, sds))(dst2d)


def _bn_tc(s0, s1, c0, c1, h, gamma, beta):
    def body(s0_ref, s1_ref, c0_ref, c1_ref, h_ref, g_ref, b_ref, o_ref):
        def asm(a_ref):
            return jnp.concatenate(
                [a_ref[0:_QN, :], a_ref[_QR:_QR + _QN, :]], axis=0)

        y = jnp.concatenate([asm(s0_ref), asm(s1_ref)], axis=0)
        cnt = jnp.concatenate(
            [c0_ref[0:_QN, 0:1], c0_ref[_QR:_QR + _QN, 0:1],
             c1_ref[0:_QN, 0:1], c1_ref[_QR:_QR + _QN, 0:1]], axis=0)
        y = y / jnp.maximum(cnt, 1.0)
        mu = jnp.mean(y, axis=0, keepdims=True)
        d = y - mu
        var = jnp.mean(d * d, axis=0, keepdims=True)
        o_ref[...] = d * lax.rsqrt(var + 1e-5) * g_ref[...] + b_ref[...] + h_ref[...]

    return pl.pallas_call(
        body,
        out_shape=jax.ShapeDtypeStruct((N, D), jnp.float32),
    )(s0, s1, c0, c1, h, gamma, beta)


def _prep(Wf, Ws, bf, bs):
    wu = jnp.concatenate([Wf[:, :D].T, Ws[:, :D].T], axis=1)
    wv = jnp.concatenate([Wf[:, D:2 * D].T, Ws[:, D:2 * D].T], axis=1)
    we = jnp.concatenate([Wf[:, 2 * D:].T, Ws[:, 2 * D:].T], axis=1)
    be = jnp.concatenate([bf, bs]).reshape(1, Z2)
    return wu, wv, we, be


def _layer(h, et, src, dst, ix01, ix23, c0, c1, w, gamma, beta):
    wu, wv = w[0], w[1]
    tu, tv = _proj_tc(h, wu, wv)
    gu, gv = _sc_gather(tu, tv, src, dst)
    msg = _msg_tc(gu, gv, et)
    s0 = _sc_scatter(msg, ix01)
    s1 = _sc_scatter(msg, ix23)
    return _bn_tc(s0, s1, c0, c1, h, gamma.reshape(1, D), beta.reshape(1, D))


def kernel(x, edge_index, edge_attr, Wf1, bf1, Ws1, bs1, g1, be1,
           Wf2, bf2, Ws2, bs2, g2, be2):
    src = edge_index[0]
    dst = edge_index[1]
    w1 = _prep(Wf1, Ws1, bf1, bs1)
    w2 = _prep(Wf2, Ws2, bf2, bs2)

    et1 = _eproj_tc(edge_attr, w1[2], w1[3])
    et2 = _eproj_tc(edge_attr, w2[2], w2[3])

    ix01, ix23 = _idx_tc(dst.reshape(E // 128, 128))
    ix01 = ix01.reshape(2 * E)
    ix23 = ix23.reshape(2 * E)

    ones = jnp.ones((E, D), jnp.float32)
    c0 = _sc_scatter(ones, ix01)
    c1 = _sc_scatter(ones, ix23)

    h1 = _layer(x, et1, src, dst, ix01, ix23, c0, c1, w1, g1, be1)
    h2 = _layer(h1, et2, src, dst, ix01, ix23, c0, c1, w2, g2, be2)
    return h2
